# baseline (speedup 1.0000x reference)
import functools

import jax
import jax.numpy as jnp
from jax import lax
from jax.experimental import pallas as pl
from jax.experimental.pallas import tpu as pltpu
from jax.experimental.pallas import tpu_sc as plsc

_N_FIELDS = 26
_VOCAB = 100000
_EMBED = 32
_BATCH = 4096
_FLAT = _BATCH * _N_FIELDS
_CHUNK = 128


def _sc_gather(feat_flat, tables_flat):
    info = plsc.get_sparse_core_info()
    nc, ns = info.num_cores, info.num_subcores
    nw = nc * ns
    per_tile = _FLAT // nw
    chunks = per_tile // _CHUNK
    nvecs = per_tile // 16

    mesh = plsc.VectorSubcoreMesh(core_axis_name="c", subcore_axis_name="s")

    @functools.partial(
        pl.kernel,
        mesh=mesh,
        out_type=jax.ShapeDtypeStruct((_FLAT, _EMBED), jnp.float32),
        scratch_types=[
            pltpu.VMEM((per_tile,), jnp.int32),
            pltpu.VMEM((per_tile, _EMBED), jnp.float32),
            pltpu.SemaphoreType.DMA,
        ],
        compiler_params=pltpu.CompilerParams(use_tc_tiling_on_sc=False),
    )
    def gather_kernel(feat_hbm, tbl_hbm, out_hbm, idx_v, rows_v, sem):
        wid = lax.axis_index("s") * nc + lax.axis_index("c")
        base = wid * per_tile
        pltpu.sync_copy(feat_hbm.at[pl.ds(base, per_tile)], idx_v)

        def vec_body(t, _):
            v = idx_v[pl.ds(t * 16, 16)]
            v = jnp.clip(v, 0, _VOCAB - 1)
            pos = t * 16 + lax.iota(jnp.int32, 16)
            idx_v[pl.ds(t * 16, 16)] = v + (pos % _N_FIELDS) * _VOCAB
            return 0

        lax.fori_loop(0, nvecs, vec_body, 0)

        copies = [
            pltpu.async_copy(
                tbl_hbm.at[idx_v.at[pl.ds(j * _CHUNK, _CHUNK)]],
                rows_v.at[pl.ds(j * _CHUNK, _CHUNK)],
                sem,
            )
            for j in range(chunks)
        ]
        for c in copies:
            c.wait()

        pltpu.sync_copy(rows_v, out_hbm.at[pl.ds(base, per_tile)])

    return gather_kernel(feat_flat, tables_flat)


def _tc_mlp(x, gamma, beta, w1, b1, w2, b2, w3, b3):
    def body(x_ref, g_ref, be_ref, w1_ref, b1_ref, w2_ref, b2_ref, w3_ref,
             b3_ref, o_ref):
        xv = x_ref[...]
        inv_n = 1.0 / xv.shape[0]
        mean = jnp.sum(xv, axis=0, keepdims=True) * inv_n
        ex2 = jnp.sum(xv * xv, axis=0, keepdims=True) * inv_n
        var = ex2 - mean * mean
        scale = g_ref[...] * lax.rsqrt(var + 1e-5)
        shift = be_ref[...] - mean * scale
        xn = xv * scale + shift
        h = lax.dot_general(xn, w1_ref[...], (((1,), (1,)), ((), ())),
                            preferred_element_type=jnp.float32)
        h = jnp.maximum(h + b1_ref[...], 0.0)
        h = lax.dot_general(h, w2_ref[...], (((1,), (1,)), ((), ())),
                            preferred_element_type=jnp.float32)
        h = jnp.maximum(h + b2_ref[...], 0.0)
        logits = lax.dot_general(h, w3_ref[...], (((1,), (1,)), ((), ())),
                                 preferred_element_type=jnp.float32)
        o_ref[...] = jax.nn.sigmoid(logits + b3_ref[0])

    n_in = 9
    w3_pad = jnp.zeros((8, w3.shape[1]), w3.dtype).at[0].set(w3[0])
    out = pl.pallas_call(
        body,
        out_shape=jax.ShapeDtypeStruct((_BATCH, 8), jnp.float32),
        in_specs=[
            pl.BlockSpec(memory_space=pltpu.SMEM) if i == n_in - 1
            else pl.BlockSpec(memory_space=pltpu.VMEM)
            for i in range(n_in)
        ],
    )(x, gamma.reshape(1, -1), beta.reshape(1, -1), w1, b1.reshape(1, -1),
      w2, b2.reshape(1, -1), w3_pad, b3)
    return out[:, 0]


def kernel(features, tables, gamma, beta, w1, b1, w2, b2, w3, b3):
    feat_flat = features.reshape(_FLAT)
    tables_flat = tables.reshape(_N_FIELDS * _VOCAB, _EMBED)
    rows = _sc_gather(feat_flat, tables_flat)
    x = rows.reshape(_BATCH, _N_FIELDS * _EMBED)
    out = _tc_mlp(x, gamma, beta, w1, b1, w2, b2, w3, b3)
    return out.reshape(_BATCH)
